# baseline (speedup 1.0000x reference)
import dataclasses
import functools

import jax
import jax.numpy as jnp
from jax import lax
from jax.experimental import pallas as pl
from jax.experimental.pallas import tpu as pltpu
from jax.experimental.pallas import tpu_sc as plsc

_NW = 32
_LANES = 16


def _sc_gather_body(rpw, s, k_l, k_f, k_d,
                    table, table_pk, nbidx, cellidx,
                    out_cur, out_l, out_f, out_d,
                    idx_v, cidx_v, curbuf, bufa, bufb, acc, sema, semb):
    wid = lax.axis_index("s") * 2 + lax.axis_index("c")
    base = wid * rpw
    sp = s // 2
    k_tot = k_l + k_f + k_d

    pltpu.sync_copy(nbidx.at[pl.ds(wid * k_tot * rpw, k_tot * rpw)], idx_v)
    pltpu.sync_copy(cellidx.at[pl.ds(base, rpw)], cidx_v)

    pltpu.sync_copy(table.at[cidx_v], curbuf)
    pltpu.sync_copy(curbuf, out_cur.at[pl.ds(base, rpw)])

    def add_from(buf):
        @pl.loop(0, rpw)
        def _(r):
            for c in range(0, sp, _LANES):
                av = plsc.bitcast(acc[r, pl.ds(c, _LANES)], jnp.bfloat16)
                bv = plsc.bitcast(buf[r, pl.ds(c, _LANES)], jnp.bfloat16)
                acc[r, pl.ds(c, _LANES)] = plsc.bitcast(av + bv, jnp.int32)

    def start(j, buf, sem):
        pltpu.async_copy(table_pk.at[idx_v.at[pl.ds(j * rpw, rpw)]], buf, sem)

    def wait(buf, sem):
        pltpu.make_async_copy(table_pk.at[idx_v.at[pl.ds(0, rpw)]], buf,
                              sem).wait()

    zeros = jnp.zeros((_LANES,), jnp.int32)

    phases = ((0, k_l, out_l), (k_l, k_l + k_f, out_f),
              (k_l + k_f, k_l + k_f + k_d, out_d))
    for j0, j1, out_ref in phases:
        npairs = (j1 - j0) // 2

        @pl.loop(0, rpw)
        def _(r):
            for c in range(0, sp, _LANES):
                acc[r, pl.ds(c, _LANES)] = zeros

        start(j0, bufa, sema)
        start(j0 + 1, bufb, semb)

        @pl.loop(0, npairs)
        def _(p):
            j = j0 + 2 * p
            wait(bufa, sema)
            add_from(bufa)

            @pl.when(j + 2 < j1)
            def _():
                start(j + 2, bufa, sema)

            wait(bufb, semb)
            add_from(bufb)

            @pl.when(j + 3 < j1)
            def _():
                start(j + 3, bufb, semb)

        pltpu.sync_copy(acc, out_ref.at[pl.ds(base, rpw)])


def _sc_gather_sums(table, table_pk, nb_idx_t, cell_idx_r, k_l, k_f, k_d):
    nw, k_tot, rpw = nb_idx_t.shape
    n, s = table.shape
    b = nw * rpw
    mesh = plsc.VectorSubcoreMesh(core_axis_name="c", subcore_axis_name="s")
    out_sum = jax.ShapeDtypeStruct((b, s // 2), jnp.int32)

    cp = pltpu.CompilerParams()
    if "needs_layout_passes" in pltpu.CompilerParams.__dataclass_fields__:
        cp = dataclasses.replace(cp, needs_layout_passes=False)
    kern = pl.kernel(
        functools.partial(_sc_gather_body, rpw, s, k_l, k_f, k_d),
        out_type=[jax.ShapeDtypeStruct((b, s), jnp.float32),
                  out_sum, out_sum, out_sum],
        mesh=mesh,
        compiler_params=cp,
        scratch_types=[
            pltpu.VMEM((k_tot * rpw,), jnp.int32),
            pltpu.VMEM((rpw,), jnp.int32),
            pltpu.VMEM((rpw, s), jnp.float32),
            pltpu.VMEM((rpw, s // 2), jnp.int32),
            pltpu.VMEM((rpw, s // 2), jnp.int32),
            pltpu.VMEM((rpw, s // 2), jnp.int32),
            pltpu.SemaphoreType.DMA,
            pltpu.SemaphoreType.DMA,
        ],
    )
    return kern(table, table_pk, nb_idx_t.reshape(-1), cell_idx_r.reshape(-1))


def _tc_moe_body(nh, s, scales, cur_ref, suml_ref, sumf_ref, sumd_ref,
                 w1l_ref, w1f_ref, w1d_ref,
                 w2l_ref, w2f_ref, w2d_ref,
                 b1l_ref, b1f_ref, b1d_ref,
                 b2l_ref, b2f_ref, b2d_ref,
                 wg_ref, bg_ref,
                 out_ref,
                 gates_ref, curbf_ref, aggl_ref, aggf_ref, aggd_ref,
                 accl_ref, accf_ref, accd_ref):
    h = pl.program_id(0)

    @pl.when(h == 0)
    def _prologue():
        curbf_ref[...] = cur_ref[...].astype(jnp.bfloat16)
        aggl_ref[...] = (suml_ref[...].astype(jnp.float32)
                         * scales[0]).astype(jnp.bfloat16)
        aggf_ref[...] = (sumf_ref[...].astype(jnp.float32)
                         * scales[1]).astype(jnp.bfloat16)
        aggd_ref[...] = (sumd_ref[...].astype(jnp.float32)
                         * scales[2]).astype(jnp.bfloat16)
        logits = jnp.dot(curbf_ref[...], wg_ref[...],
                         preferred_element_type=jnp.float32) + bg_ref[...]
        m = jnp.max(logits, axis=-1, keepdims=True)
        p = jnp.exp(logits - m)
        gates_ref[...] = p / jnp.sum(p, axis=-1, keepdims=True)

    experts = (
        (aggl_ref, w1l_ref, w2l_ref, b1l_ref, accl_ref),
        (aggf_ref, w1f_ref, w2f_ref, b1f_ref, accf_ref),
        (aggd_ref, w1d_ref, w2d_ref, b1d_ref, accd_ref),
    )
    for agg_ref, w1_ref, w2_ref, b1_ref, acc_ref in experts:
        x1 = jnp.dot(curbf_ref[...], w1_ref[0:s, :],
                     preferred_element_type=jnp.float32)
        x1 = x1 + jnp.dot(agg_ref[...], w1_ref[s:2 * s, :],
                          preferred_element_type=jnp.float32)
        x1 = x1 + b1_ref[...]
        hact = jnp.maximum(x1, 0.0).astype(jnp.bfloat16)
        part = jnp.dot(hact, w2_ref[...],
                       preferred_element_type=jnp.float32)

        @pl.when(h == 0)
        def _init(acc_ref=acc_ref, part=part):
            acc_ref[...] = part

        @pl.when(h != 0)
        def _accum(acc_ref=acc_ref, part=part):
            acc_ref[...] += part

    @pl.when(h == nh - 1)
    def _epilogue():
        combined = cur_ref[...]
        for e, (_, _, _, _, acc_ref) in enumerate(experts):
            b2_ref = (b2l_ref, b2f_ref, b2d_ref)[e]
            combined = combined + gates_ref[:, e:e + 1] * jnp.tanh(
                acc_ref[...] + b2_ref[...])
        out_ref[...] = combined


def _tc_moe(cur, suml, sumf, sumd, scales,
            w1l, w1f, w1d, w2l, w2f, w2d,
            b1l, b1f, b1d, b2l, b2f, b2d, wg_pad, bg_pad):
    b, s = cur.shape
    hdim = w1l.shape[1]
    hc = min(512, hdim)
    nh = hdim // hc

    full = lambda shape: pl.BlockSpec(shape, lambda h: (0, 0))
    w1_spec = pl.BlockSpec((2 * s, hc), lambda h: (0, h))
    w2_spec = pl.BlockSpec((hc, s), lambda h: (h, 0))
    b1_spec = pl.BlockSpec((1, hc), lambda h: (0, h))

    return pl.pallas_call(
        functools.partial(_tc_moe_body, nh, s, scales),
        grid=(nh,),
        in_specs=[
            full((b, s)), full((b, s)), full((b, s)), full((b, s)),
            w1_spec, w1_spec, w1_spec,
            w2_spec, w2_spec, w2_spec,
            b1_spec, b1_spec, b1_spec,
            full((1, s)), full((1, s)), full((1, s)),
            full((s, 128)), full((1, 128)),
        ],
        out_specs=pl.BlockSpec((b, s), lambda h: (0, 0)),
        out_shape=jax.ShapeDtypeStruct((b, s), jnp.float32),
        scratch_shapes=[
            pltpu.VMEM((b, 128), jnp.float32),
            pltpu.VMEM((b, s), jnp.bfloat16),
            pltpu.VMEM((b, s), jnp.bfloat16),
            pltpu.VMEM((b, s), jnp.bfloat16),
            pltpu.VMEM((b, s), jnp.bfloat16),
            pltpu.VMEM((b, s), jnp.float32),
            pltpu.VMEM((b, s), jnp.float32),
            pltpu.VMEM((b, s), jnp.float32),
        ],
        compiler_params=pltpu.CompilerParams(
            dimension_semantics=("arbitrary",)),
    )(cur, suml, sumf, sumd, w1l, w1f, w1d, w2l, w2f, w2d,
      b1l, b1f, b1d,
      b2l.reshape(1, s), b2f.reshape(1, s), b2d.reshape(1, s),
      wg_pad, bg_pad)


def kernel(cell_indices, full_lattice_states, local_idx, functional_idx,
           distant_idx,
           W1_local, b1_local, W2_local, b2_local,
           W1_functional, b1_functional, W2_functional, b2_functional,
           W1_distant, b1_distant, W2_distant, b2_distant,
           Wg, bg):
    n, s = full_lattice_states.shape
    b = cell_indices.shape[0]
    k_l, k_f, k_d = (local_idx.shape[1], functional_idx.shape[1],
                     distant_idx.shape[1])
    k_tot = k_l + k_f + k_d
    scales = (1.0 / k_l, 1.0 / k_f, 1.0 / k_d)

    wg_pad = jnp.pad(Wg, ((0, 0), (0, 128 - Wg.shape[1]))).astype(jnp.bfloat16)
    bg_pad = jnp.pad(bg, (0, 128 - bg.shape[0]),
                     constant_values=-1e30).reshape(1, 128)

    nb = jnp.concatenate([local_idx, functional_idx, distant_idx], axis=1)
    nb = jnp.asarray(nb, jnp.int32)
    cell = jnp.asarray(cell_indices, jnp.int32)

    w1l_bf, w1f_bf, w1d_bf = (w.astype(jnp.bfloat16)
                              for w in (W1_local, W1_functional, W1_distant))
    w2l_bf, w2f_bf, w2d_bf = (w.astype(jnp.bfloat16)
                              for w in (W2_local, W2_functional, W2_distant))
    table_pk = lax.bitcast_convert_type(
        full_lattice_states.astype(jnp.bfloat16).reshape(n, s // 2, 2),
        jnp.int32)

    nchunks = 2
    bc = b // nchunks
    rpw = bc // _NW
    outs = []
    for c in range(nchunks):
        rows = slice(c * bc, (c + 1) * bc)
        nb_t = nb[rows].reshape(_NW, rpw, k_tot).transpose(0, 2, 1)
        cell_r = cell[rows].reshape(_NW, rpw)
        cur, sum_l, sum_f, sum_d = _sc_gather_sums(
            full_lattice_states, table_pk, nb_t, cell_r, k_l, k_f, k_d)
        sum_l, sum_f, sum_d = (
            lax.bitcast_convert_type(x, jnp.bfloat16).reshape(bc, s)
            for x in (sum_l, sum_f, sum_d))
        outs.append(_tc_moe(cur, sum_l, sum_f, sum_d, scales,
                            w1l_bf, w1f_bf, w1d_bf,
                            w2l_bf, w2f_bf, w2d_bf,
                            b1_local.reshape(1, -1),
                            b1_functional.reshape(1, -1),
                            b1_distant.reshape(1, -1),
                            b2_local, b2_functional, b2_distant,
                            wg_pad, bg_pad))
    return jnp.concatenate(outs, axis=0)


# speedup vs baseline: 2.2974x; 2.2974x over previous
import dataclasses
import functools

import jax
import jax.numpy as jnp
from jax import lax
from jax.experimental import pallas as pl
from jax.experimental.pallas import tpu as pltpu
from jax.experimental.pallas import tpu_sc as plsc

_NW = 32
_LANES = 16


def _sc_gather_body(rpw, s, k_l, k_f, k_d,
                    table, table_pk, nbidx, cellidx,
                    out_cur, out_l, out_f, out_d,
                    idx_v, cidx_v, curbuf, bufa, bufb, acc, sema, semb):
    wid = lax.axis_index("s") * 2 + lax.axis_index("c")
    base = wid * rpw
    sp = s // 2
    k_tot = k_l + k_f + k_d

    pltpu.sync_copy(nbidx.at[pl.ds(wid * k_tot * rpw, k_tot * rpw)], idx_v)
    pltpu.sync_copy(cellidx.at[pl.ds(base, rpw)], cidx_v)

    pltpu.sync_copy(table.at[cidx_v], curbuf)
    pltpu.sync_copy(curbuf, out_cur.at[pl.ds(base, rpw)])

    def add_from(buf):
        @pl.loop(0, rpw)
        def _(r):
            for c in range(0, sp, _LANES):
                av = plsc.bitcast(acc[r, pl.ds(c, _LANES)], jnp.bfloat16)
                bv = plsc.bitcast(buf[r, pl.ds(c, _LANES)], jnp.bfloat16)
                acc[r, pl.ds(c, _LANES)] = plsc.bitcast(av + bv, jnp.int32)

    def start(j, buf, sem):
        pltpu.async_copy(table_pk.at[idx_v.at[pl.ds(j * rpw, rpw)]], buf, sem)

    def wait(buf, sem):
        pltpu.make_async_copy(table_pk.at[idx_v.at[pl.ds(0, rpw)]], buf,
                              sem).wait()

    zeros = jnp.zeros((_LANES,), jnp.int32)

    phases = ((0, k_l, out_l), (k_l, k_l + k_f, out_f),
              (k_l + k_f, k_l + k_f + k_d, out_d))
    for j0, j1, out_ref in phases:
        npairs = (j1 - j0) // 2

        @pl.loop(0, rpw)
        def _(r):
            for c in range(0, sp, _LANES):
                acc[r, pl.ds(c, _LANES)] = zeros

        start(j0, bufa, sema)
        start(j0 + 1, bufb, semb)

        @pl.loop(0, npairs)
        def _(p):
            j = j0 + 2 * p
            wait(bufa, sema)
            add_from(bufa)

            @pl.when(j + 2 < j1)
            def _():
                start(j + 2, bufa, sema)

            wait(bufb, semb)
            add_from(bufb)

            @pl.when(j + 3 < j1)
            def _():
                start(j + 3, bufb, semb)

        pltpu.sync_copy(acc, out_ref.at[pl.ds(base, rpw)])


def _sc_gather_sums(table, table_pk, nb_idx_t, cell_idx_r, k_l, k_f, k_d):
    nw, k_tot, rpw = nb_idx_t.shape
    n, s = table.shape
    b = nw * rpw
    mesh = plsc.VectorSubcoreMesh(core_axis_name="c", subcore_axis_name="s")
    out_sum = jax.ShapeDtypeStruct((b, s // 2), jnp.int32)

    cp = pltpu.CompilerParams()
    if "needs_layout_passes" in pltpu.CompilerParams.__dataclass_fields__:
        cp = dataclasses.replace(cp, needs_layout_passes=False)
    kern = pl.kernel(
        functools.partial(_sc_gather_body, rpw, s, k_l, k_f, k_d),
        out_type=[jax.ShapeDtypeStruct((b, s), jnp.float32),
                  out_sum, out_sum, out_sum],
        mesh=mesh,
        compiler_params=cp,
        scratch_types=[
            pltpu.VMEM((k_tot * rpw,), jnp.int32),
            pltpu.VMEM((rpw,), jnp.int32),
            pltpu.VMEM((rpw, s), jnp.float32),
            pltpu.VMEM((rpw, s // 2), jnp.int32),
            pltpu.VMEM((rpw, s // 2), jnp.int32),
            pltpu.VMEM((rpw, s // 2), jnp.int32),
            pltpu.SemaphoreType.DMA,
            pltpu.SemaphoreType.DMA,
        ],
    )
    return kern(table, table_pk, nb_idx_t.reshape(-1), cell_idx_r.reshape(-1))


def _pack_body(sp, x_ref, o_ref):
    xf = x_ref[...].astype(jnp.bfloat16).astype(jnp.float32)
    bits = lax.bitcast_convert_type(xf, jnp.int32)
    o_ref[...] = lax.shift_right_logical(bits[:, :sp], 16) | bits[:, sp:]


def _pack_table(table):
    n, s = table.shape
    sp = s // 2
    blk = 2048
    return pl.pallas_call(
        functools.partial(_pack_body, sp),
        grid=(n // blk,),
        in_specs=[pl.BlockSpec((blk, s), lambda i: (i, 0))],
        out_specs=pl.BlockSpec((blk, sp), lambda i: (i, 0)),
        out_shape=jax.ShapeDtypeStruct((n, sp), jnp.int32),
    )(table)


def _tc_moe_body(nh, s, scales, cur_ref, suml_ref, sumf_ref, sumd_ref,
                 w1l_ref, w1f_ref, w1d_ref,
                 w2l_ref, w2f_ref, w2d_ref,
                 b1l_ref, b1f_ref, b1d_ref,
                 b2l_ref, b2f_ref, b2d_ref,
                 wg_ref, bg_ref,
                 out_ref,
                 gates_ref, curbf_ref, aggl_ref, aggf_ref, aggd_ref,
                 accl_ref, accf_ref, accd_ref):
    h = pl.program_id(0)

    sp = s // 2

    @pl.when(h == 0)
    def _prologue():
        curbf_ref[...] = cur_ref[...].astype(jnp.bfloat16)
        for sum_ref, agg_ref, scale in ((suml_ref, aggl_ref, scales[0]),
                                        (sumf_ref, aggf_ref, scales[1]),
                                        (sumd_ref, aggd_ref, scales[2])):
            u = sum_ref[...]
            lo = lax.bitcast_convert_type(lax.shift_left(u, 16), jnp.float32)
            hi = lax.bitcast_convert_type(
                jnp.bitwise_and(u, jnp.int32(-65536)), jnp.float32)
            agg_ref[:, 0:sp] = (lo * scale).astype(jnp.bfloat16)
            agg_ref[:, sp:s] = (hi * scale).astype(jnp.bfloat16)
        logits = jnp.dot(curbf_ref[...], wg_ref[...],
                         preferred_element_type=jnp.float32) + bg_ref[...]
        m = jnp.max(logits, axis=-1, keepdims=True)
        p = jnp.exp(logits - m)
        gates_ref[...] = p / jnp.sum(p, axis=-1, keepdims=True)

    experts = (
        (aggl_ref, w1l_ref, w2l_ref, b1l_ref, accl_ref),
        (aggf_ref, w1f_ref, w2f_ref, b1f_ref, accf_ref),
        (aggd_ref, w1d_ref, w2d_ref, b1d_ref, accd_ref),
    )
    for agg_ref, w1_ref, w2_ref, b1_ref, acc_ref in experts:
        x1 = jnp.dot(curbf_ref[...], w1_ref[0:s, :],
                     preferred_element_type=jnp.float32)
        x1 = x1 + jnp.dot(agg_ref[...], w1_ref[s:2 * s, :],
                          preferred_element_type=jnp.float32)
        x1 = x1 + b1_ref[...]
        hact = jnp.maximum(x1, 0.0).astype(jnp.bfloat16)
        part = jnp.dot(hact, w2_ref[...],
                       preferred_element_type=jnp.float32)

        @pl.when(h == 0)
        def _init(acc_ref=acc_ref, part=part):
            acc_ref[...] = part

        @pl.when(h != 0)
        def _accum(acc_ref=acc_ref, part=part):
            acc_ref[...] += part

    @pl.when(h == nh - 1)
    def _epilogue():
        combined = cur_ref[...]
        for e, (_, _, _, _, acc_ref) in enumerate(experts):
            b2_ref = (b2l_ref, b2f_ref, b2d_ref)[e]
            combined = combined + gates_ref[:, e:e + 1] * jnp.tanh(
                acc_ref[...] + b2_ref[...])
        out_ref[...] = combined


def _tc_moe(cur, suml, sumf, sumd, scales,
            w1l, w1f, w1d, w2l, w2f, w2d,
            b1l, b1f, b1d, b2l, b2f, b2d, wg_pad, bg_pad):
    b, s = cur.shape
    hdim = w1l.shape[1]
    hc = min(512, hdim)
    nh = hdim // hc

    full = lambda shape: pl.BlockSpec(shape, lambda h: (0, 0))
    w1_spec = pl.BlockSpec((2 * s, hc), lambda h: (0, h))
    w2_spec = pl.BlockSpec((hc, s), lambda h: (h, 0))
    b1_spec = pl.BlockSpec((1, hc), lambda h: (0, h))

    return pl.pallas_call(
        functools.partial(_tc_moe_body, nh, s, scales),
        grid=(nh,),
        in_specs=[
            full((b, s)),
            full((b, s // 2)), full((b, s // 2)), full((b, s // 2)),
            w1_spec, w1_spec, w1_spec,
            w2_spec, w2_spec, w2_spec,
            b1_spec, b1_spec, b1_spec,
            full((1, s)), full((1, s)), full((1, s)),
            full((s, 128)), full((1, 128)),
        ],
        out_specs=pl.BlockSpec((b, s), lambda h: (0, 0)),
        out_shape=jax.ShapeDtypeStruct((b, s), jnp.float32),
        scratch_shapes=[
            pltpu.VMEM((b, 128), jnp.float32),
            pltpu.VMEM((b, s), jnp.bfloat16),
            pltpu.VMEM((b, s), jnp.bfloat16),
            pltpu.VMEM((b, s), jnp.bfloat16),
            pltpu.VMEM((b, s), jnp.bfloat16),
            pltpu.VMEM((b, s), jnp.float32),
            pltpu.VMEM((b, s), jnp.float32),
            pltpu.VMEM((b, s), jnp.float32),
        ],
        compiler_params=pltpu.CompilerParams(
            dimension_semantics=("arbitrary",)),
    )(cur, suml, sumf, sumd, w1l, w1f, w1d, w2l, w2f, w2d,
      b1l, b1f, b1d,
      b2l.reshape(1, s), b2f.reshape(1, s), b2d.reshape(1, s),
      wg_pad, bg_pad)


def kernel(cell_indices, full_lattice_states, local_idx, functional_idx,
           distant_idx,
           W1_local, b1_local, W2_local, b2_local,
           W1_functional, b1_functional, W2_functional, b2_functional,
           W1_distant, b1_distant, W2_distant, b2_distant,
           Wg, bg):
    n, s = full_lattice_states.shape
    b = cell_indices.shape[0]
    k_l, k_f, k_d = (local_idx.shape[1], functional_idx.shape[1],
                     distant_idx.shape[1])
    k_tot = k_l + k_f + k_d
    scales = (1.0 / k_l, 1.0 / k_f, 1.0 / k_d)

    wg_pad = jnp.pad(Wg, ((0, 0), (0, 128 - Wg.shape[1]))).astype(jnp.bfloat16)
    bg_pad = jnp.pad(bg, (0, 128 - bg.shape[0]),
                     constant_values=-1e30).reshape(1, 128)

    nb = jnp.concatenate([local_idx, functional_idx, distant_idx], axis=1)
    nb = jnp.asarray(nb, jnp.int32)
    cell = jnp.asarray(cell_indices, jnp.int32)

    w1l_bf, w1f_bf, w1d_bf = (w.astype(jnp.bfloat16)
                              for w in (W1_local, W1_functional, W1_distant))
    w2l_bf, w2f_bf, w2d_bf = (w.astype(jnp.bfloat16)
                              for w in (W2_local, W2_functional, W2_distant))
    table_pk = _pack_table(full_lattice_states)

    nchunks = 2
    bc = b // nchunks
    rpw = bc // _NW
    outs = []
    for c in range(nchunks):
        rows = slice(c * bc, (c + 1) * bc)
        nb_t = nb[rows].reshape(_NW, rpw, k_tot).transpose(0, 2, 1)
        cell_r = cell[rows].reshape(_NW, rpw)
        cur, sum_l, sum_f, sum_d = _sc_gather_sums(
            full_lattice_states, table_pk, nb_t, cell_r, k_l, k_f, k_d)
        outs.append(_tc_moe(cur, sum_l, sum_f, sum_d, scales,
                            w1l_bf, w1f_bf, w1d_bf,
                            w2l_bf, w2f_bf, w2d_bf,
                            b1_local.reshape(1, -1),
                            b1_functional.reshape(1, -1),
                            b1_distant.reshape(1, -1),
                            b2_local, b2_functional, b2_distant,
                            wg_pad, bg_pad))
    return jnp.concatenate(outs, axis=0)
